# baseline (speedup 1.0000x reference)
import jax
import jax.numpy as jnp
from jax import lax
from jax.experimental import pallas as pl
from jax.experimental.pallas import tpu as pltpu

_P = 1568
_K = 784
_CH = 112
_S_SCALE = float(2 ** 34)


def _body(seq_ref, table_ref, thr_ref, mi_ref, rep_ref, idx_ref, mask_ref):
    b = pl.program_id(0)
    v_row = seq_ref[pl.ds(b, 1), :]
    rep_ref[0] = jnp.broadcast_to(v_row, (_K, _P))

    mi_s = mi_ref[b, 0]
    row = jnp.zeros((1, _P), jnp.float32)
    for m in range(4):
        row = row + jnp.where(mi_s == m, 1.0, 0.0) * table_ref[m : m + 1, :]
    mask_ref[0] = 1.0 - row

    s_row = v_row * _S_SCALE
    ones_row = jnp.ones((1, _CH), jnp.bfloat16)

    s_colfull = jnp.transpose(s_row, (1, 0))

    rank = jnp.zeros((1, _P), jnp.float32)
    for c in range(_P // _CH):
        jb = c * _CH
        s_col = s_colfull[jb : jb + _CH, :]
        d = s_col - s_row
        ind = jnp.where(d > thr_ref[c], 1.0, 0.0).astype(jnp.bfloat16)
        rank = rank + jax.lax.dot_general(
            ones_row, ind, (((1,), (0,)), ((), ())),
            preferred_element_type=jnp.float32,
        )

    iq = lax.broadcasted_iota(jnp.int32, (_P, 2), 0)
    lane = lax.broadcasted_iota(jnp.int32, (_P, 2), 1)
    qs = jnp.where(lane == 0, iq // 8, iq % 8).astype(jnp.bfloat16)
    r_col = lax.broadcasted_iota(jnp.int32, (_CH, 1), 0).astype(jnp.float32)
    for rc in range(_K // _CH):
        rb = rc * _CH
        eq = rank == (r_col + jnp.float32(rb))
        ind2 = jnp.where(eq, 1.0, 0.0).astype(jnp.bfloat16)
        col2 = jax.lax.dot_general(
            ind2, qs, (((1,), (0,)), ((), ())),
            preferred_element_type=jnp.float32,
        )
        col = col2[:, 0:1] * 8.0 + col2[:, 1:2]
        idx_ref[0, pl.ds(rb, _CH), :] = col.astype(jnp.int32)


def kernel(x, mask_shape, train_mask):
    B = x.shape[0]
    key = jax.random.key(42)
    k1, k2 = jax.random.split(key)
    mask_index = jax.random.randint(k1, (B, 1), 0, train_mask.shape[0])
    seq_logits = jax.random.uniform(k2, (B, _P), dtype=jnp.float32)

    tme = train_mask.astype(jnp.float32).reshape(4, 8, 1, 2, 1, 2)
    table = jnp.broadcast_to(tme, (4, 8, 7, 2, 7, 2)).reshape(4, _P)

    nch = _P // _CH
    thr = (
        lax.broadcasted_iota(jnp.int32, (nch, _CH, _P), 0) * _CH
        + lax.broadcasted_iota(jnp.int32, (nch, _CH, _P), 1)
        - lax.broadcasted_iota(jnp.int32, (nch, _CH, _P), 2)
    ).astype(jnp.float32)

    rep, idx3, mask3 = pl.pallas_call(
        _body,
        grid=(B,),
        in_specs=[
            pl.BlockSpec((B, _P), lambda b: (0, 0)),
            pl.BlockSpec((4, _P), lambda b: (0, 0)),
            pl.BlockSpec((_P // _CH, _CH, _P), lambda b: (0, 0, 0)),
            pl.BlockSpec((B, 1), lambda b: (0, 0), memory_space=pltpu.SMEM),
        ],
        out_specs=[
            pl.BlockSpec((1, _K, _P), lambda b: (b, 0, 0)),
            pl.BlockSpec((1, _K, 1), lambda b: (b, 0, 0)),
            pl.BlockSpec((1, 1, _P), lambda b: (b, 0, 0)),
        ],
        out_shape=[
            jax.ShapeDtypeStruct((B, _K, _P), jnp.float32),
            jax.ShapeDtypeStruct((B, _K, 1), jnp.int32),
            jax.ShapeDtypeStruct((B, 1, _P), jnp.float32),
        ],
    )(seq_logits, table, thr, mask_index)

    return rep, idx3.reshape(B, _K), mask3.reshape(B, _P)


# speedup vs baseline: 1.1594x; 1.1594x over previous
import jax
import jax.numpy as jnp
from jax import lax
from jax.experimental import pallas as pl
from jax.experimental.pallas import tpu as pltpu

_P = 1568
_K = 784
_CH = 112
_S_SCALE = float(2 ** 34)


def _body(seq_ref, table_ref, mi_ref, rep_ref, idx_ref, mask_ref, buf_ref, sem):
    b = pl.program_id(0)
    v_row = seq_ref[pl.ds(b, 1), :]

    buf_ref[...] = jnp.broadcast_to(v_row, (_CH, _P))
    copies = [
        pltpu.make_async_copy(
            buf_ref, rep_ref.at[b, pl.ds(t * _CH, _CH), :], sem
        )
        for t in range(_K // _CH)
    ]
    for cp in copies:
        cp.start()

    mi_s = mi_ref[b, 0]
    row = jnp.zeros((1, _P), jnp.float32)
    for m in range(4):
        row = row + jnp.where(mi_s == m, 1.0, 0.0) * table_ref[m : m + 1, :]
    mask_ref[0] = 1.0 - row

    s_row = v_row * _S_SCALE
    ir = (
        lax.broadcasted_iota(jnp.int32, (_CH, _P), 1)
        - lax.broadcasted_iota(jnp.int32, (_CH, _P), 0)
    ).astype(jnp.float32)
    ones_row = jnp.ones((1, _CH), jnp.bfloat16)

    s_colfull = jnp.transpose(s_row, (1, 0))

    rank = jnp.zeros((1, _P), jnp.float32)
    for c in range(_P // _CH):
        jb = c * _CH
        s_col = s_colfull[jb : jb + _CH, :]
        d = (s_col - s_row) + ir
        ind = jnp.where(d > jnp.float32(jb), 1.0, 0.0).astype(jnp.bfloat16)
        rank = rank + jax.lax.dot_general(
            ones_row, ind, (((1,), (0,)), ((), ())),
            preferred_element_type=jnp.float32,
        )

    iq = lax.broadcasted_iota(jnp.int32, (_P, 2), 0)
    lane = lax.broadcasted_iota(jnp.int32, (_P, 2), 1)
    qs = jnp.where(lane == 0, iq // 8, iq % 8).astype(jnp.bfloat16)
    r_col = lax.broadcasted_iota(jnp.int32, (_CH, 1), 0).astype(jnp.float32)
    for rc in range(_K // _CH):
        rb = rc * _CH
        eq = rank == (r_col + jnp.float32(rb))
        ind2 = jnp.where(eq, 1.0, 0.0).astype(jnp.bfloat16)
        col2 = jax.lax.dot_general(
            ind2, qs, (((1,), (0,)), ((), ())),
            preferred_element_type=jnp.float32,
        )
        col = col2[:, 0:1] * 8.0 + col2[:, 1:2]
        idx_ref[0, pl.ds(rb, _CH), :] = col.astype(jnp.int32)

    for cp in copies:
        cp.wait()


def kernel(x, mask_shape, train_mask):
    B = x.shape[0]
    key = jax.random.key(42)
    k1, k2 = jax.random.split(key)
    mask_index = jax.random.randint(k1, (B, 1), 0, train_mask.shape[0])
    seq_logits = jax.random.uniform(k2, (B, _P), dtype=jnp.float32)

    tme = train_mask.astype(jnp.float32).reshape(4, 8, 1, 2, 1, 2)
    table = jnp.broadcast_to(tme, (4, 8, 7, 2, 7, 2)).reshape(4, _P)


    rep, idx3, mask3 = pl.pallas_call(
        _body,
        grid=(B,),
        in_specs=[
            pl.BlockSpec((B, _P), lambda b: (0, 0)),
            pl.BlockSpec((4, _P), lambda b: (0, 0)),
            pl.BlockSpec((B, 1), lambda b: (0, 0), memory_space=pltpu.SMEM),
        ],
        out_specs=[
            pl.BlockSpec(memory_space=pl.ANY),
            pl.BlockSpec((1, _K, 1), lambda b: (b, 0, 0)),
            pl.BlockSpec((1, 1, _P), lambda b: (b, 0, 0)),
        ],
        out_shape=[
            jax.ShapeDtypeStruct((B, _K, _P), jnp.float32),
            jax.ShapeDtypeStruct((B, _K, 1), jnp.int32),
            jax.ShapeDtypeStruct((B, 1, _P), jnp.float32),
        ],
        scratch_shapes=[
            pltpu.VMEM((_CH, _P), jnp.float32),
            pltpu.SemaphoreType.DMA,
        ],
    )(seq_logits, table, mask_index)

    return rep, idx3.reshape(B, _K), mask3.reshape(B, _P)


# speedup vs baseline: 1.1685x; 1.0078x over previous
import jax
import jax.numpy as jnp
from jax import lax
from jax.experimental import pallas as pl
from jax.experimental.pallas import tpu as pltpu

_P = 1568
_K = 784
_CH = 112
_S_SCALE = float(2 ** 34)


def _body(seq_ref, table_ref, mi_ref, rep_ref, idx_ref, mask_ref, buf_ref, sem):
    b = pl.program_id(0)
    v_row = seq_ref[pl.ds(b, 1), :]

    buf_ref[...] = jnp.broadcast_to(v_row, (_CH, _P))
    copies = [
        pltpu.make_async_copy(
            buf_ref, rep_ref.at[b, pl.ds(t * _CH, _CH), :], sem
        )
        for t in range(_K // _CH)
    ]
    for cp in copies:
        cp.start()

    mi_s = mi_ref[b, 0]
    row = jnp.zeros((1, _P), jnp.float32)
    for m in range(4):
        row = row + jnp.where(mi_s == m, 1.0, 0.0) * table_ref[m : m + 1, :]
    mask_ref[0] = 1.0 - row

    s_row = v_row * _S_SCALE
    ir = (
        lax.broadcasted_iota(jnp.int32, (_CH, _P), 1)
        - lax.broadcasted_iota(jnp.int32, (_CH, _P), 0)
    ).astype(jnp.float32)
    ones_row = jnp.ones((1, _CH), jnp.float32)

    s_colfull = jnp.transpose(s_row, (1, 0))

    rank = jnp.zeros((1, _P), jnp.float32)
    for c in range(_P // _CH):
        jb = c * _CH
        s_col = s_colfull[jb : jb + _CH, :]
        d = (s_col - s_row) + ir
        ind = jnp.where(d > jnp.float32(jb), 1.0, 0.0)
        rank = rank + jax.lax.dot_general(
            ones_row, ind, (((1,), (0,)), ((), ())),
            preferred_element_type=jnp.float32,
        )

    iq = lax.broadcasted_iota(jnp.int32, (_P, 2), 0)
    lane = lax.broadcasted_iota(jnp.int32, (_P, 2), 1)
    qs = jnp.where(lane == 0, iq // 8, iq % 8).astype(jnp.float32)
    r_col = lax.broadcasted_iota(jnp.int32, (_CH, 1), 0).astype(jnp.float32)
    for rc in range(_K // _CH):
        rb = rc * _CH
        eq = rank == (r_col + jnp.float32(rb))
        ind2 = jnp.where(eq, 1.0, 0.0)
        col2 = jax.lax.dot_general(
            ind2, qs, (((1,), (0,)), ((), ())),
            preferred_element_type=jnp.float32,
        )
        col = col2[:, 0:1] * 8.0 + col2[:, 1:2]
        idx_ref[0, pl.ds(rb, _CH), :] = col.astype(jnp.int32)

    for cp in copies:
        cp.wait()


def kernel(x, mask_shape, train_mask):
    B = x.shape[0]
    key = jax.random.key(42)
    k1, k2 = jax.random.split(key)
    mask_index = jax.random.randint(k1, (B, 1), 0, train_mask.shape[0])
    seq_logits = jax.random.uniform(k2, (B, _P), dtype=jnp.float32)

    tme = train_mask.astype(jnp.float32).reshape(4, 8, 1, 2, 1, 2)
    table = jnp.broadcast_to(tme, (4, 8, 7, 2, 7, 2)).reshape(4, _P)


    rep, idx3, mask3 = pl.pallas_call(
        _body,
        grid=(B,),
        in_specs=[
            pl.BlockSpec((B, _P), lambda b: (0, 0)),
            pl.BlockSpec((4, _P), lambda b: (0, 0)),
            pl.BlockSpec((B, 1), lambda b: (0, 0), memory_space=pltpu.SMEM),
        ],
        out_specs=[
            pl.BlockSpec(memory_space=pl.ANY),
            pl.BlockSpec((1, _K, 1), lambda b: (b, 0, 0)),
            pl.BlockSpec((1, 1, _P), lambda b: (b, 0, 0)),
        ],
        out_shape=[
            jax.ShapeDtypeStruct((B, _K, _P), jnp.float32),
            jax.ShapeDtypeStruct((B, _K, 1), jnp.int32),
            jax.ShapeDtypeStruct((B, 1, _P), jnp.float32),
        ],
        scratch_shapes=[
            pltpu.VMEM((_CH, _P), jnp.float32),
            pltpu.SemaphoreType.DMA,
        ],
    )(seq_logits, table, mask_index)

    return rep, idx3.reshape(B, _K), mask3.reshape(B, _P)
